# baseline (speedup 1.0000x reference)
import functools

import jax
import jax.numpy as jnp
from jax import lax
from jax.experimental import pallas as pl
from jax.experimental.pallas import tpu as pltpu
from jax.experimental.pallas import tpu_sc as plsc

N = 10000
N1 = 10240
E_RAW = 320000
E_REAL = E_RAW + N
CH = 96
NW = 32
NCH = 108
EPW = NCH * CH
E1 = EPW * NW
BR = 512
NBLK = N1 // BR
RPS = N1 // 16

f32 = jnp.float32
bf16 = jnp.bfloat16
i32 = jnp.int32


def _scramble_bf16(m):
    br, w = m.shape
    g = w // 32
    m = m.reshape(br, g, 2, 16).swapaxes(2, 3).reshape(br, w)
    return m.astype(bf16)


def _full(shape):
    return pl.BlockSpec(shape, lambda i: tuple(0 for _ in shape))


def _rows(shape):
    return pl.BlockSpec(shape, lambda i: (i,) + tuple(0 for _ in shape[1:]))


def _prep_pre(x, W, a_s, a_d):
    din, dout = W.shape

    def body(x_ref, w_ref, as_ref, ad_ref, m_ref, uv_ref, g_ref, sm):
        i = pl.program_id(0)
        xb = x_ref[...]
        w = w_ref[...]
        wu = jnp.dot(w, as_ref[...], preferred_element_type=f32)
        wv = jnp.dot(w, ad_ref[...], preferred_element_type=f32)
        u = jnp.dot(xb, wu, preferred_element_type=f32)
        v = jnp.dot(xb, wv, preferred_element_type=f32)
        ones = jnp.ones((BR, 1), f32)
        zeros = jnp.zeros((BR, 31), f32)
        m_ref[...] = _scramble_bf16(jnp.concatenate([xb, ones, zeros],
                                                    axis=1))
        uv_ref[...] = jnp.concatenate([u, v], axis=1).T
        bm = jnp.max(u)

        @pl.when(i == 0)
        def _():
            sm[0] = bm

        @pl.when(i > 0)
        def _():
            sm[0] = jnp.maximum(sm[0], bm)
        g_ref[...] = jnp.full((1, 16), sm[0], f32)

    return pl.pallas_call(
        body,
        grid=(NBLK,),
        in_specs=[_rows((BR, din)), _full((din, dout)),
                  _full((dout, 1)), _full((dout, 1))],
        out_specs=[_rows((BR, 160)),
                   pl.BlockSpec((2, BR), lambda i: (0, i)),
                   pl.BlockSpec((1, 16), lambda i: (0, 0))],
        out_shape=[jax.ShapeDtypeStruct((N1, 160), bf16),
                   jax.ShapeDtypeStruct((2, N1), f32),
                   jax.ShapeDtypeStruct((1, 16), f32)],
        scratch_shapes=[pltpu.SMEM((1,), f32)],
    )(x, W, a_s, a_d)


def _prep_post(x, W, a_s, a_d, widths):
    din, dout = W.shape

    def body(x_ref, w_ref, as_ref, ad_ref, *refs):
        sm = refs[-1]
        g_ref = refs[-2]
        uv_ref = refs[-3]
        m_refs = refs[:-3]
        i = pl.program_id(0)
        h = jnp.dot(x_ref[...], w_ref[...], preferred_element_type=f32)
        u = jnp.dot(h, as_ref[...], preferred_element_type=f32)
        v = jnp.dot(h, ad_ref[...], preferred_element_type=f32)
        col = 0
        msg0 = dout - sum(widths[1:])
        for k, w_k in enumerate(widths):
            dm = w_k if k > 0 else msg0
            piece = h[:, col:col + dm]
            col += dm
            if k == 0:
                piece = jnp.concatenate(
                    [piece, jnp.ones((BR, 1), f32),
                     jnp.zeros((BR, w_k - msg0 - 1), f32)], axis=1)
            m_refs[k][...] = _scramble_bf16(piece)
        uv_ref[...] = jnp.concatenate([u, v], axis=1).T
        bm = jnp.max(u)

        @pl.when(i == 0)
        def _():
            sm[0] = bm

        @pl.when(i > 0)
        def _():
            sm[0] = jnp.maximum(sm[0], bm)
        g_ref[...] = jnp.full((1, 16), sm[0], f32)

    return pl.pallas_call(
        body,
        grid=(NBLK,),
        in_specs=[_rows((BR, din)), _full((din, dout)),
                  _full((dout, 1)), _full((dout, 1))],
        out_specs=[_rows((BR, w)) for w in widths]
                  + [pl.BlockSpec((2, BR), lambda i: (0, i)),
                     pl.BlockSpec((1, 16), lambda i: (0, 0))],
        out_shape=[jax.ShapeDtypeStruct((N1, w), bf16) for w in widths]
                  + [jax.ShapeDtypeStruct((2, N1), f32),
                     jax.ShapeDtypeStruct((1, 16), f32)],
        scratch_shapes=[pltpu.SMEM((1,), f32)],
    )(x, W, a_s, a_d)


def _finish_matmul(accp, W, b, relu):
    din, dout = W.shape

    def body(a_ref, w_ref, b_ref, o_ref):
        s = a_ref[0] + a_ref[1]
        den = jnp.maximum(s[:, 128:129], 1e-30)
        g = s[:, :din] / den
        o = jnp.dot(g, w_ref[...], preferred_element_type=f32) + b_ref[...]
        o_ref[...] = jnp.maximum(o, 0.0) if relu else o

    return pl.pallas_call(
        body,
        grid=(NBLK,),
        in_specs=[pl.BlockSpec((2, BR, 144), lambda i: (0, i, 0)),
                  _full((din, dout)), _full((1, dout))],
        out_specs=_rows((BR, dout)),
        out_shape=jax.ShapeDtypeStruct((N1, dout), f32),
    )(accp, W, b)


def _finish_add(accps, b, widths, dout, relu, logsm):
    def body(*refs):
        a_refs = refs[:len(widths)]
        b_ref, o_ref = refs[-2], refs[-1]
        s0 = a_refs[0][0] + a_refs[0][1]
        den = jnp.maximum(s0[:, widths[0] - 16:widths[0] - 15], 1e-30)
        pieces = [s0[:, :widths[0] - 16]]
        for k in range(1, len(widths)):
            pieces.append(a_refs[k][0] + a_refs[k][1])
        g = jnp.concatenate(pieces, axis=1) if len(pieces) > 1 else pieces[0]
        t = g / den + b_ref[...]
        if relu:
            t = jnp.maximum(t, 0.0)
        if logsm:
            m = jnp.max(t, axis=1, keepdims=True)
            t = t - (m + jnp.log(jnp.sum(jnp.exp(t - m), axis=1,
                                         keepdims=True)))
        o_ref[...] = t

    return pl.pallas_call(
        body,
        grid=(NBLK,),
        in_specs=[pl.BlockSpec((2, BR, w), lambda i: (0, i, 0))
                  for w in widths] + [_full((1, dout))],
        out_specs=_rows((BR, dout)),
        out_shape=jax.ShapeDtypeStruct((N1, dout), f32),
    )(*accps, b)


@functools.lru_cache(maxsize=None)
def _make_sc_edge_pass(d_bf, d_acc):
    mesh = plsc.VectorSubcoreMesh(core_axis_name="c", subcore_axis_name="s")

    @functools.partial(
        pl.kernel,
        out_type=jax.ShapeDtypeStruct((2, N1, d_acc), f32),
        mesh=mesh,
        compiler_params=pltpu.CompilerParams(needs_layout_passes=False,
                                             use_tc_tiling_on_sc=False),
        scratch_types=[
            pltpu.VMEM_SHARED((N1,), f32),
            pltpu.VMEM_SHARED((N1,), f32),
            pltpu.VMEM_SHARED((N1, d_acc), f32),
            [pltpu.VMEM((CH,), i32) for _ in range(2)],
            [pltpu.VMEM((CH,), i32) for _ in range(2)],
            [pltpu.VMEM((CH,), f32) for _ in range(2)],
            [pltpu.VMEM((CH,), f32) for _ in range(2)],
            [pltpu.VMEM((CH,), f32) for _ in range(2)],
            [pltpu.VMEM((CH, d_bf), bf16) for _ in range(2)],
            pltpu.VMEM((CH, d_acc), f32),
            pltpu.VMEM((16, d_acc), f32),
            pltpu.VMEM((16,), f32),
            [pltpu.SemaphoreType.DMA for _ in range(10)],
        ],
    )
    def sc_pass(m_hbm, src_hbm, dst_hbm, uv_hbm, g_hbm, out_hbm,
                u_sh, v_sh, acc, sv, dv, ub, vb, pv, rows, scaled,
                zbuf, gbuf, sems):
        cid = lax.axis_index("c")
        sid = lax.axis_index("s")
        nv = d_acc // 16
        ssv, sdv, sub_s, svb, srw = (sems[0:2], sems[2:4], sems[4:6],
                                     sems[6:8], sems[8:10])

        @pl.when(sid == 0)
        def _():
            pltpu.sync_copy(uv_hbm.at[0], u_sh)
            pltpu.sync_copy(uv_hbm.at[1], v_sh)

        def zrow(e, carry):
            for k in range(nv):
                zbuf[e, pl.ds(16 * k, 16)] = jnp.zeros((16,), f32)
            return carry
        lax.fori_loop(0, 16, zrow, 0)
        row0 = sid * RPS

        def zacc(k, carry):
            pltpu.sync_copy(zbuf, acc.at[pl.ds(row0 + 16 * k, 16)])
            return carry
        lax.fori_loop(0, RPS // 16, zacc, 0)

        pltpu.sync_copy(g_hbm.at[0], gbuf)
        gmax = jnp.max(gbuf[...])

        plsc.subcore_barrier()

        ebase = (cid * 16 + sid) * EPW

        def w1_issue(ch, s):
            b = ebase + ch * CH
            pltpu.async_copy(src_hbm.at[pl.ds(b, CH)], sv[s], ssv[s])
            pltpu.async_copy(dst_hbm.at[pl.ds(b, CH)], dv[s], sdv[s])

        def w1_wait(ch, s):
            b = ebase + ch * CH
            pltpu.make_async_copy(src_hbm.at[pl.ds(b, CH)], sv[s],
                                  ssv[s]).wait()
            pltpu.make_async_copy(dst_hbm.at[pl.ds(b, CH)], dv[s],
                                  sdv[s]).wait()

        def w2_issue(s):
            pltpu.async_copy(u_sh.at[sv[s]], ub[s], sub_s[s])
            pltpu.async_copy(v_sh.at[dv[s]], vb[s], svb[s])
            pltpu.async_copy(m_hbm.at[sv[s]], rows[s], srw[s])

        def w2_wait(s):
            pltpu.make_async_copy(u_sh.at[sv[s]], ub[s], sub_s[s]).wait()
            pltpu.make_async_copy(v_sh.at[dv[s]], vb[s], svb[s]).wait()
            pltpu.make_async_copy(m_hbm.at[sv[s]], rows[s], srw[s]).wait()

        w1_issue(0, 0)
        w1_wait(0, 0)
        w2_issue(0)
        w1_issue(1, 1)

        def pair_body(g, carry):
            for s in range(2):
                ch = 2 * g + s
                o = 1 - s
                w2_wait(s)

                @plsc.parallel_loop(0, CH // 16)
                def _(j):
                    ug = ub[s][pl.ds(16 * j, 16)]
                    vg = vb[s][pl.ds(16 * j, 16)]
                    zz = ug + vg
                    e = jnp.maximum(zz, 0.2 * zz)
                    zub = gmax + vg
                    cc = jnp.maximum(zub, 0.2 * zub)
                    pv[s][pl.ds(16 * j, 16)] = jnp.exp(e - cc)

                @plsc.parallel_loop(0, CH, unroll=2)
                def _(e_i):
                    pb = plsc.load_gather(pv[s], [jnp.full((16,), e_i, i32)])
                    for g2 in range(d_bf // 32):
                        raw = rows[s][e_i, pl.ds(32 * g2, 32)]
                        a, b = plsc.unpack(
                            raw, format=plsc.PackFormat.INTERLEAVED,
                            preferred_element_type=f32)
                        scaled[e_i, pl.ds(32 * g2, 16)] = a * pb
                        if 32 * g2 + 32 <= d_acc:
                            scaled[e_i, pl.ds(32 * g2 + 16, 16)] = b * pb

                pltpu.sync_copy(scaled, acc.at[dv[s]], add=True)

                @pl.when(ch + 2 < NCH)
                def _():
                    w1_issue(ch + 2, s)

                @pl.when(ch + 1 < NCH)
                def _():
                    w1_wait(ch + 1, o)
                    w2_issue(o)
            return carry
        lax.fori_loop(0, NCH // 2, pair_body, 0)

        plsc.subcore_barrier()
        for k in range(RPS // 128):
            r0 = row0 + k * 128
            pltpu.sync_copy(acc.at[pl.ds(r0, 128)],
                            out_hbm.at[cid, pl.ds(r0, 128)])

    return sc_pass


def kernel(x, edge_index, W1, a1_src, a1_dst, b1, W2, a2_src, a2_dst, b2,
           W3, a3_src, a3_dst, b3, W4, a4_src, a4_dst, b4):
    loops = jnp.arange(N, dtype=edge_index.dtype)
    pad = jnp.full((E1 - E_REAL,), N, i32)
    src = jnp.concatenate([edge_index[0], loops, pad])
    dst = jnp.concatenate([edge_index[1], loops, pad])

    x0 = jnp.zeros((N1, 128), f32).at[:N].set(x.astype(f32))

    def col(a):
        return a.astype(f32).reshape(-1, 1)

    def row(b):
        return b.astype(f32).reshape(1, -1)

    sc128 = _make_sc_edge_pass(160, 144)
    sc_l4 = _make_sc_edge_pass(32, 32)

    m, uv, g = _prep_pre(x0, W1, col(a1_src), col(a1_dst))
    accp = sc128(m, src, dst, uv, g)
    h = _finish_matmul(accp, W1, row(b1), relu=True)

    m, uv, g = _prep_pre(h, W2, col(a2_src), col(a2_dst))
    accp = sc128(m, src, dst, uv, g)
    h = _finish_matmul(accp, W2, row(b2), relu=True)

    bf3 = (160, 128, 128, 128)
    acc3 = (144, 128, 128, 128)
    outs = _prep_post(h, W3, col(a3_src), col(a3_dst), bf3)
    ms, uv, g = outs[:-2], outs[-2], outs[-1]
    accps = [_make_sc_edge_pass(wb, wa)(mk, src, dst, uv, g)
             for wb, wa, mk in zip(bf3, acc3, ms)]
    h = _finish_add(accps, row(b3), acc3, 512, relu=True, logsm=False)

    outs = _prep_post(h, W4, col(a4_src), col(a4_dst), (32,))
    m, uv, g = outs[0], outs[1], outs[2]
    accp = sc_l4(m, src, dst, uv, g)
    out = _finish_add([accp], row(b4), (32,), 16, relu=False, logsm=True)

    return out[:N]


# speedup vs baseline: 1.4712x; 1.4712x over previous
import functools

import jax
import jax.numpy as jnp
from jax import lax
from jax.experimental import pallas as pl
from jax.experimental.pallas import tpu as pltpu
from jax.experimental.pallas import tpu_sc as plsc

N = 10000
N1 = 10240
E_RAW = 320000
E_REAL = E_RAW + N
CH = 96
NW = 32
NCH = 108
EPW = NCH * CH
E1 = EPW * NW
BR = 512
NBLK = N1 // BR
RPS = N1 // 16

f32 = jnp.float32
bf16 = jnp.bfloat16
i32 = jnp.int32


def _scr(w):
    import numpy as np
    p = np.arange(w).reshape(-1, 2, 16).swapaxes(1, 2).reshape(-1)
    return jnp.asarray(p, i32)


def _spread_pos(widths, nmsg):
    import numpy as np
    offs = np.cumsum([0] + list(widths))[:-1]
    m0 = nmsg - sum(widths[1:])
    pos = []
    for k, w_k in enumerate(widths):
        mk = m0 if k == 0 else w_k
        for c in range(mk):
            g, j = divmod(c, 32)
            raw = 32 * g + (2 * j if j < 16 else 2 * (j - 16) + 1)
            pos.append(offs[k] + raw)
    g, j = divmod(m0, 32)
    rawo = 32 * g + (2 * j if j < 16 else 2 * (j - 16) + 1)
    ones = np.zeros((1, int(sum(widths))), np.float32)
    ones[0, offs[0] + rawo] = 1.0
    return jnp.asarray(np.array(pos), i32), jnp.asarray(ones)


def _full(shape):
    return pl.BlockSpec(shape, lambda i: tuple(0 for _ in shape))


def _rows(shape):
    return pl.BlockSpec(shape, lambda i: (i,) + tuple(0 for _ in shape[1:]))


def _prep_pre(x, W, a_s, a_d):
    din, dout = W.shape

    def body(x_ref, w_ref, as_ref, ad_ref, m_ref, uv_ref, g_ref, sm):
        i = pl.program_id(0)
        xb = x_ref[...]
        w = w_ref[...]
        wu = jnp.dot(w, as_ref[...], preferred_element_type=f32)
        wv = jnp.dot(w, ad_ref[...], preferred_element_type=f32)
        u = jnp.dot(xb, wu, preferred_element_type=f32)
        v = jnp.dot(xb, wv, preferred_element_type=f32)
        ones = jnp.ones((BR, 1), f32)
        zeros = jnp.zeros((BR, 31), f32)
        m_ref[...] = jnp.concatenate([xb, ones, zeros],
                                     axis=1).astype(bf16)
        uv_ref[...] = jnp.concatenate([u, v], axis=1).T
        bm = jnp.max(u)

        @pl.when(i == 0)
        def _():
            sm[0] = bm

        @pl.when(i > 0)
        def _():
            sm[0] = jnp.maximum(sm[0], bm)
        g_ref[...] = jnp.full((1, 16), sm[0], f32)

    return pl.pallas_call(
        body,
        grid=(NBLK,),
        in_specs=[_rows((BR, din)), _full((din, dout)),
                  _full((dout, 1)), _full((dout, 1))],
        out_specs=[_rows((BR, 160)),
                   pl.BlockSpec((2, BR), lambda i: (0, i)),
                   pl.BlockSpec((1, 16), lambda i: (0, 0))],
        out_shape=[jax.ShapeDtypeStruct((N1, 160), bf16),
                   jax.ShapeDtypeStruct((2, N1), f32),
                   jax.ShapeDtypeStruct((1, 16), f32)],
        scratch_shapes=[pltpu.SMEM((1,), f32)],
    )(x, W, a_s, a_d)


def _prep_post(x, W_pre, a_ps, a_pd, onesrow, widths):
    din, wt = W_pre.shape

    def body(x_ref, w_ref, as_ref, ad_ref, on_ref, *refs):
        sm = refs[-1]
        g_ref = refs[-2]
        uv_ref = refs[-3]
        m_refs = refs[:-3]
        i = pl.program_id(0)
        hm = jnp.dot(x_ref[...], w_ref[...], preferred_element_type=f32)
        u = jnp.dot(hm, as_ref[...], preferred_element_type=f32)
        v = jnp.dot(hm, ad_ref[...], preferred_element_type=f32)
        hm = hm + on_ref[...]
        col = 0
        for k, w_k in enumerate(widths):
            m_refs[k][...] = hm[:, col:col + w_k].astype(bf16)
            col += w_k
        uv_ref[...] = jnp.concatenate([u, v], axis=1).T
        bm = jnp.max(u)

        @pl.when(i == 0)
        def _():
            sm[0] = bm

        @pl.when(i > 0)
        def _():
            sm[0] = jnp.maximum(sm[0], bm)
        g_ref[...] = jnp.full((1, 16), sm[0], f32)

    return pl.pallas_call(
        body,
        grid=(NBLK,),
        in_specs=[_rows((BR, din)), _full((din, wt)),
                  _full((wt, 1)), _full((wt, 1)), _full((1, wt))],
        out_specs=[_rows((BR, w)) for w in widths]
                  + [pl.BlockSpec((2, BR), lambda i: (0, i)),
                     pl.BlockSpec((1, 16), lambda i: (0, 0))],
        out_shape=[jax.ShapeDtypeStruct((N1, w), bf16) for w in widths]
                  + [jax.ShapeDtypeStruct((2, N1), f32),
                     jax.ShapeDtypeStruct((1, 16), f32)],
        scratch_shapes=[pltpu.SMEM((1,), f32)],
    )(x, W_pre, a_ps, a_pd, onesrow)


def _finish_matmul(accp, W, b, relu):
    din, dout = W.shape

    def body(a_ref, w_ref, b_ref, o_ref):
        s = a_ref[0] + a_ref[1]
        den = jnp.maximum(s[:, 128:129], 1e-30)
        g = s[:, :din] / den
        o = jnp.dot(g, w_ref[...], preferred_element_type=f32) + b_ref[...]
        o_ref[...] = jnp.maximum(o, 0.0) if relu else o

    return pl.pallas_call(
        body,
        grid=(NBLK,),
        in_specs=[pl.BlockSpec((2, BR, 144), lambda i: (0, i, 0)),
                  _full((din, dout)), _full((1, dout))],
        out_specs=_rows((BR, dout)),
        out_shape=jax.ShapeDtypeStruct((N1, dout), f32),
    )(accp, W, b)


def _finish_add(accps, b, widths, dout, relu, logsm):
    def body(*refs):
        a_refs = refs[:len(widths)]
        b_ref, o_ref = refs[-2], refs[-1]
        s0 = a_refs[0][0] + a_refs[0][1]
        den = jnp.maximum(s0[:, widths[0] - 16:widths[0] - 15], 1e-30)
        pieces = [s0[:, :widths[0] - 16]]
        for k in range(1, len(widths)):
            pieces.append(a_refs[k][0] + a_refs[k][1])
        g = jnp.concatenate(pieces, axis=1) if len(pieces) > 1 else pieces[0]
        t = g / den + b_ref[...]
        if relu:
            t = jnp.maximum(t, 0.0)
        if logsm:
            m = jnp.max(t, axis=1, keepdims=True)
            t = t - (m + jnp.log(jnp.sum(jnp.exp(t - m), axis=1,
                                         keepdims=True)))
        o_ref[...] = t

    return pl.pallas_call(
        body,
        grid=(NBLK,),
        in_specs=[pl.BlockSpec((2, BR, w), lambda i: (0, i, 0))
                  for w in widths] + [_full((1, dout))],
        out_specs=_rows((BR, dout)),
        out_shape=jax.ShapeDtypeStruct((N1, dout), f32),
    )(*accps, b)


@functools.lru_cache(maxsize=None)
def _make_sc_edge_pass(d_bf, d_acc):
    mesh = plsc.VectorSubcoreMesh(core_axis_name="c", subcore_axis_name="s")

    @functools.partial(
        pl.kernel,
        out_type=jax.ShapeDtypeStruct((2, N1, d_acc), f32),
        mesh=mesh,
        compiler_params=pltpu.CompilerParams(needs_layout_passes=False,
                                             use_tc_tiling_on_sc=False),
        scratch_types=[
            pltpu.VMEM_SHARED((N1,), f32),
            pltpu.VMEM_SHARED((N1,), f32),
            pltpu.VMEM_SHARED((N1, d_acc), f32),
            [pltpu.VMEM((CH,), i32) for _ in range(2)],
            [pltpu.VMEM((CH,), i32) for _ in range(2)],
            [pltpu.VMEM((CH,), f32) for _ in range(2)],
            [pltpu.VMEM((CH,), f32) for _ in range(2)],
            [pltpu.VMEM((CH,), f32) for _ in range(2)],
            [pltpu.VMEM((CH, d_bf), bf16) for _ in range(2)],
            pltpu.VMEM((CH, d_acc), f32),
            pltpu.VMEM((16, d_acc), f32),
            pltpu.VMEM((16,), f32),
            [pltpu.SemaphoreType.DMA for _ in range(10)],
        ],
    )
    def sc_pass(m_hbm, src_hbm, dst_hbm, uv_hbm, g_hbm, out_hbm,
                u_sh, v_sh, acc, sv, dv, ub, vb, pv, rows, scaled,
                zbuf, gbuf, sems):
        cid = lax.axis_index("c")
        sid = lax.axis_index("s")
        nv = d_acc // 16
        ssv, sdv, sub_s, svb, srw = (sems[0:2], sems[2:4], sems[4:6],
                                     sems[6:8], sems[8:10])

        @pl.when(sid == 0)
        def _():
            pltpu.sync_copy(uv_hbm.at[0], u_sh)
            pltpu.sync_copy(uv_hbm.at[1], v_sh)

        def zrow(e, carry):
            for k in range(nv):
                zbuf[e, pl.ds(16 * k, 16)] = jnp.zeros((16,), f32)
            return carry
        lax.fori_loop(0, 16, zrow, 0)
        row0 = sid * RPS

        def zacc(k, carry):
            pltpu.sync_copy(zbuf, acc.at[pl.ds(row0 + 16 * k, 16)])
            return carry
        lax.fori_loop(0, RPS // 16, zacc, 0)

        pltpu.sync_copy(g_hbm.at[0], gbuf)
        gmax = jnp.max(gbuf[...])

        plsc.subcore_barrier()

        ebase = (cid * 16 + sid) * EPW

        def w1_issue(ch, s):
            b = ebase + ch * CH
            pltpu.async_copy(src_hbm.at[pl.ds(b, CH)], sv[s], ssv[s])
            pltpu.async_copy(dst_hbm.at[pl.ds(b, CH)], dv[s], sdv[s])

        def w1_wait(ch, s):
            b = ebase + ch * CH
            pltpu.make_async_copy(src_hbm.at[pl.ds(b, CH)], sv[s],
                                  ssv[s]).wait()
            pltpu.make_async_copy(dst_hbm.at[pl.ds(b, CH)], dv[s],
                                  sdv[s]).wait()

        def w2_issue(s):
            pltpu.async_copy(u_sh.at[sv[s]], ub[s], sub_s[s])
            pltpu.async_copy(v_sh.at[dv[s]], vb[s], svb[s])
            pltpu.async_copy(m_hbm.at[sv[s]], rows[s], srw[s])

        def w2_wait(s):
            pltpu.make_async_copy(u_sh.at[sv[s]], ub[s], sub_s[s]).wait()
            pltpu.make_async_copy(v_sh.at[dv[s]], vb[s], svb[s]).wait()
            pltpu.make_async_copy(m_hbm.at[sv[s]], rows[s], srw[s]).wait()

        w1_issue(0, 0)
        w1_wait(0, 0)
        w2_issue(0)
        w1_issue(1, 1)

        def pair_body(g, carry):
            for s in range(2):
                ch = 2 * g + s
                o = 1 - s
                w2_wait(s)

                @plsc.parallel_loop(0, CH // 16)
                def _(j):
                    ug = ub[s][pl.ds(16 * j, 16)]
                    vg = vb[s][pl.ds(16 * j, 16)]
                    zz = ug + vg
                    e = jnp.maximum(zz, 0.2 * zz)
                    zub = gmax + vg
                    cc = jnp.maximum(zub, 0.2 * zub)
                    pv[s][pl.ds(16 * j, 16)] = jnp.exp(e - cc)

                @plsc.parallel_loop(0, CH, unroll=2)
                def _(e_i):
                    pb = plsc.load_gather(pv[s], [jnp.full((16,), e_i, i32)])
                    for g2 in range(d_bf // 32):
                        raw = rows[s][e_i, pl.ds(32 * g2, 32)]
                        a, b = plsc.unpack(
                            raw, format=plsc.PackFormat.INTERLEAVED,
                            preferred_element_type=f32)
                        scaled[e_i, pl.ds(32 * g2, 16)] = a * pb
                        if 32 * g2 + 32 <= d_acc:
                            scaled[e_i, pl.ds(32 * g2 + 16, 16)] = b * pb

                pltpu.sync_copy(scaled, acc.at[dv[s]], add=True)

                @pl.when(ch + 2 < NCH)
                def _():
                    w1_issue(ch + 2, s)

                @pl.when(ch + 1 < NCH)
                def _():
                    w1_wait(ch + 1, o)
                    w2_issue(o)
            return carry
        lax.fori_loop(0, NCH // 2, pair_body, 0)

        plsc.subcore_barrier()
        for k in range(RPS // 128):
            r0 = row0 + k * 128
            pltpu.sync_copy(acc.at[pl.ds(r0, 128)],
                            out_hbm.at[cid, pl.ds(r0, 128)])

    return sc_pass


def kernel(x, edge_index, W1, a1_src, a1_dst, b1, W2, a2_src, a2_dst, b2,
           W3, a3_src, a3_dst, b3, W4, a4_src, a4_dst, b4):
    loops = jnp.arange(N, dtype=edge_index.dtype)
    pad = jnp.full((E1 - E_REAL,), N, i32)
    src = jnp.concatenate([edge_index[0], loops, pad])
    dst = jnp.concatenate([edge_index[1], loops, pad])

    x0 = jnp.zeros((N1, 128), f32).at[:N].set(x.astype(f32))

    def col(a):
        return a.astype(f32).reshape(-1, 1)

    def row(b):
        return b.astype(f32).reshape(1, -1)

    s128 = _scr(128)
    s1024 = _scr(1024)
    x0s = jnp.take(x0, s128, axis=1)

    sc128 = _make_sc_edge_pass(160, 144)
    sc_l4 = _make_sc_edge_pass(32, 32)

    m, uv, g = _prep_pre(x0s, jnp.take(W1, s128, axis=0),
                         col(a1_src), col(a1_dst))
    accp = sc128(m, src, dst, uv, g)
    h = _finish_matmul(accp, jnp.take(W1, s128, axis=1),
                       row(b1)[:, s128], relu=True)

    m, uv, g = _prep_pre(h, jnp.take(W2, s128, axis=0),
                         col(a2_src), col(a2_dst))
    accp = sc128(m, src, dst, uv, g)
    h = _finish_matmul(accp, jnp.take(W2, s1024, axis=1),
                       row(b2)[:, s1024], relu=True)

    bf3 = (160, 128, 128, 128)
    acc3 = (144, 128, 128, 128)
    pos3, ones3 = _spread_pos(bf3, 512)
    w3r = jnp.take(W3, s1024, axis=0)
    wpre3 = jnp.zeros((1024, sum(bf3)), f32).at[:, pos3].set(w3r)
    a3s = jnp.zeros((sum(bf3), 1), f32).at[pos3, 0].set(a3_src.astype(f32))
    a3d = jnp.zeros((sum(bf3), 1), f32).at[pos3, 0].set(a3_dst.astype(f32))
    outs = _prep_post(h, wpre3, a3s, a3d, ones3, bf3)
    ms, uv, g = outs[:-2], outs[-2], outs[-1]
    accps = [_make_sc_edge_pass(wb, wa)(mk, src, dst, uv, g)
             for wb, wa, mk in zip(bf3, acc3, ms)]
    h = _finish_add(accps, row(b3), acc3, 512, relu=True, logsm=False)

    bf4 = (32,)
    pos4, ones4 = _spread_pos(bf4, 16)
    wpre4 = jnp.zeros((512, 32), f32).at[:, pos4].set(W4.astype(f32))
    a4s = jnp.zeros((32, 1), f32).at[pos4, 0].set(a4_src.astype(f32))
    a4d = jnp.zeros((32, 1), f32).at[pos4, 0].set(a4_dst.astype(f32))
    outs = _prep_post(h, wpre4, a4s, a4d, ones4, bf4)
    m, uv, g = outs[0], outs[1], outs[2]
    accp = sc_l4(m, src, dst, uv, g)
    out = _finish_add([accp], row(b4), bf4, 16, relu=False, logsm=True)

    return out[:N]
